# Initial kernel scaffold; baseline (speedup 1.0000x reference)
import jax
import jax.numpy as jnp
from jax.experimental import pallas as pl


def kernel(source, target):
    raise NotImplementedError("write your pallas kernel here")


# speedup vs baseline: 452.0296x; 452.0296x over previous
import functools

import jax
import jax.numpy as jnp
from jax import lax
from jax.experimental import pallas as pl
from jax.experimental.pallas import tpu as pltpu
from jax.experimental.pallas import tpu_sc as plsc

H = 64
W = 64
HW = H * W
ITERATIONS = 3
RADIUS = 4
NW = 32
ROWS_PER_W = (2 * H) // NW
LANES = 16
MAGIC = 8388608.0


def _dfield_body(s_ref, t_ref, o_ref):
    c = pl.program_id(1)
    nc = pl.num_programs(1)
    diff = s_ref[0] - t_ref[0]
    part = jnp.sum(diff * diff, axis=0, keepdims=True)[None]

    @pl.when(c == 0)
    def _():
        o_ref[...] = part

    @pl.when(c != 0)
    def _():
        o_ref[...] = o_ref[...] + part

    @pl.when(c == nc - 1)
    def _():
        e2 = o_ref[0]
        pos = lax.broadcasted_iota(jnp.int32, (1, HW), 1)
        col = pos & (W - 1)
        row = pos >> 6
        left = jnp.where(col >= 1, jnp.roll(e2, 1, axis=1), 0.0)
        right = jnp.where(col <= W - 2, jnp.roll(e2, -1, axis=1), 0.0)
        sx = e2 + left + right
        up = jnp.where(row >= 1, jnp.roll(sx, W, axis=1), 0.0)
        down = jnp.where(row <= H - 2, jnp.roll(sx, -W, axis=1), 0.0)
        o_ref[0] = sx + up + down


def _dfield(source, target):
    B, C, _, _ = source.shape
    CB = 32
    s2 = source.reshape(B, C, HW)
    t2 = target.reshape(B, C, HW)
    return pl.pallas_call(
        _dfield_body,
        grid=(B, C // CB),
        in_specs=[
            pl.BlockSpec((1, CB, HW), lambda b, c: (b, c, 0)),
            pl.BlockSpec((1, CB, HW), lambda b, c: (b, c, 0)),
        ],
        out_specs=pl.BlockSpec((1, 1, HW), lambda b, c: (b, 0, 0)),
        out_shape=jax.ShapeDtypeStruct((B, 1, HW), jnp.float32),
    )(s2, t2)


def _sc_loop_body(d_hbm, u_hbm, o_hbm, out_hbm, uv, ov, dv, yb, xb, db):
    wid = lax.axis_index("s") * 2 + lax.axis_index("c")
    rows0 = wid * ROWS_PER_W
    b = wid // (NW // 2)

    pltpu.sync_copy(u_hbm.at[pl.ds(rows0 * 2 * W, ROWS_PER_W * 2 * W)], uv)
    pltpu.sync_copy(
        o_hbm.at[pl.ds(rows0 * RADIUS * ITERATIONS * 2 * W,
                       ROWS_PER_W * RADIUS * ITERATIONS * 2 * W)], ov)
    pltpu.sync_copy(d_hbm.at[pl.ds(b * HW, HW)], dv)

    iota = lax.iota(jnp.int32, LANES)
    nchunk = W // LANES

    def lg(ref, idx):
        return plsc.load_gather(ref, [idx])

    def rnd_int(v):
        return ((v + MAGIC) - MAGIC).astype(jnp.int32)

    def row_body(r, carry):
        base = r * W

        for i in range(nchunk):
            ci = iota + (base + i * LANES)
            uy = lg(uv, iota + (r * 2 * W + i * LANES))
            ux = lg(uv, iota + (r * 2 * W + W + i * LANES))
            y = uy * float(H - 1)
            x = ux * float(W - 1)
            d0 = lg(dv, rnd_int(y) * W + rnd_int(x))
            plsc.store_scatter(yb, [ci], y)
            plsc.store_scatter(xb, [ci], x)
            plsc.store_scatter(db, [ci], d0)

        def t_body(t, tc):
            for dirn in (1, -1):
                res = []
                for i in range(nchunk):
                    ci = iota + (base + i * LANES)
                    sj = ((iota + (i * LANES - dirn)) & (W - 1)) + base
                    cy, cx, cd = lg(yb, ci), lg(xb, ci), lg(db, ci)
                    sy, sx, sd = lg(yb, sj), lg(xb, sj), lg(db, sj)
                    m = sd < cd
                    res.append((ci,
                                jnp.where(m, sy, cy),
                                jnp.where(m, sx, cx),
                                jnp.where(m, sd, cd)))
                for ci, ny, nx, nd in res:
                    plsc.store_scatter(yb, [ci], ny)
                    plsc.store_scatter(xb, [ci], nx)
                    plsc.store_scatter(db, [ci], nd)

            for s in range(RADIUS):
                dr = t * RADIUS + s
                for i in range(nchunk):
                    ci = iota + (base + i * LANES)
                    ob = r * (RADIUS * ITERATIONS * 2 * W) + dr * 2 * W + i * LANES
                    offy = lg(ov, iota + ob)
                    offx = lg(ov, iota + (ob + W))
                    y, x, d0 = lg(yb, ci), lg(xb, ci), lg(db, ci)
                    ry = jnp.minimum(jnp.maximum(y + offy, 0.0), float(H - 1))
                    rx = jnp.minimum(jnp.maximum(x + offx, 0.0), float(W - 1))
                    rd = lg(dv, rnd_int(ry) * W + rnd_int(rx))
                    m = rd < d0
                    plsc.store_scatter(yb, [ci], jnp.where(m, ry, y))
                    plsc.store_scatter(xb, [ci], jnp.where(m, rx, x))
                    plsc.store_scatter(db, [ci], jnp.where(m, rd, d0))
            return tc

        lax.fori_loop(0, ITERATIONS, t_body, 0)

        for i in range(nchunk):
            ci = iota + (base + i * LANES)
            plsc.store_scatter(uv, [iota + (r * 2 * W + i * LANES)], lg(yb, ci))
            plsc.store_scatter(uv, [iota + (r * 2 * W + W + i * LANES)], lg(xb, ci))
        return carry

    lax.fori_loop(0, ROWS_PER_W, row_body, 0)

    pltpu.sync_copy(uv, out_hbm.at[pl.ds(rows0 * 2 * W, ROWS_PER_W * 2 * W)])


def _sc_loop(d_flat, u_flat, offs_flat):
    B = 2
    mesh = plsc.VectorSubcoreMesh(core_axis_name="c", subcore_axis_name="s")
    fn = functools.partial(
        pl.kernel,
        mesh=mesh,
        out_type=jax.ShapeDtypeStruct((B * H * 2 * W,), jnp.float32),
        scratch_types=[
            pltpu.VMEM((ROWS_PER_W * 2 * W,), jnp.float32),
            pltpu.VMEM((ROWS_PER_W * RADIUS * ITERATIONS * 2 * W,), jnp.float32),
            pltpu.VMEM((HW,), jnp.float32),
            pltpu.VMEM((ROWS_PER_W * W,), jnp.float32),
            pltpu.VMEM((ROWS_PER_W * W,), jnp.float32),
            pltpu.VMEM((ROWS_PER_W * W,), jnp.float32),
        ],
        compiler_params=pltpu.CompilerParams(needs_layout_passes=False),
    )(_sc_loop_body)
    return fn(d_flat, u_flat, offs_flat)


def kernel(source, target):
    B, C, _, _ = source.shape

    dfield = _dfield(source, target)

    key = jax.random.key(42)
    key, k0 = jax.random.split(key)
    u = jax.random.uniform(k0, (B, 2, H, W), dtype=jnp.float32)
    offs = []
    for _ in range(ITERATIONS):
        key, sub = jax.random.split(key)
        k = sub
        for i in range(RADIUS):
            k, s2 = jax.random.split(k)
            offs.append(jax.random.normal(s2, (B, 2, H, W), dtype=jnp.float32)
                        * (2.0 ** (-i)))
    offs = jnp.stack(offs)

    u_flat = jnp.transpose(u, (0, 2, 1, 3)).reshape(-1)
    offs_flat = jnp.transpose(offs, (1, 3, 0, 2, 4)).reshape(-1)

    out_flat = _sc_loop(dfield.reshape(-1), u_flat, offs_flat)
    return jnp.transpose(out_flat.reshape(B, H, 2, W), (0, 2, 1, 3))


# speedup vs baseline: 1045.5194x; 2.3129x over previous
import functools

import jax
import jax.numpy as jnp
from jax import lax
from jax.experimental import pallas as pl
from jax.experimental.pallas import tpu as pltpu
from jax.experimental.pallas import tpu_sc as plsc

H = 64
W = 64
HW = H * W
ITERATIONS = 3
RADIUS = 4
NW = 32
ROWS_PER_W = (2 * H) // NW
LANES = 16
MAGIC = 8388608.0


def _dfield_body(s_ref, t_ref, o_ref):
    c = pl.program_id(1)
    nc = pl.num_programs(1)
    diff = s_ref[0] - t_ref[0]
    part = jnp.sum(diff * diff, axis=0, keepdims=True)[None]

    @pl.when(c == 0)
    def _():
        o_ref[...] = part

    @pl.when(c != 0)
    def _():
        o_ref[...] = o_ref[...] + part

    @pl.when(c == nc - 1)
    def _():
        e2 = o_ref[0]
        pos = lax.broadcasted_iota(jnp.int32, (1, HW), 1)
        col = pos & (W - 1)
        row = pos >> 6
        left = jnp.where(col >= 1, jnp.roll(e2, 1, axis=1), 0.0)
        right = jnp.where(col <= W - 2, jnp.roll(e2, -1, axis=1), 0.0)
        sx = e2 + left + right
        up = jnp.where(row >= 1, jnp.roll(sx, W, axis=1), 0.0)
        down = jnp.where(row <= H - 2, jnp.roll(sx, -W, axis=1), 0.0)
        o_ref[0] = sx + up + down


def _dfield(source, target):
    B, C, _, _ = source.shape
    CB = 32
    s2 = source.reshape(B, C, HW)
    t2 = target.reshape(B, C, HW)
    return pl.pallas_call(
        _dfield_body,
        grid=(B, C // CB),
        in_specs=[
            pl.BlockSpec((1, CB, HW), lambda b, c: (b, c, 0)),
            pl.BlockSpec((1, CB, HW), lambda b, c: (b, c, 0)),
        ],
        out_specs=pl.BlockSpec((1, 1, HW), lambda b, c: (b, 0, 0)),
        out_shape=jax.ShapeDtypeStruct((B, 1, HW), jnp.float32),
    )(s2, t2)


def _sc_loop_body(d_hbm, u_hbm, o_hbm, out_hbm, uv, ov, dv, yb, xb, db):
    wid = lax.axis_index("s") * 2 + lax.axis_index("c")
    rows0 = wid * ROWS_PER_W
    b = wid // (NW // 2)

    pltpu.sync_copy(u_hbm.at[pl.ds(rows0 * 2 * W, ROWS_PER_W * 2 * W)], uv)
    pltpu.sync_copy(
        o_hbm.at[pl.ds(rows0 * RADIUS * ITERATIONS * 2 * W,
                       ROWS_PER_W * RADIUS * ITERATIONS * 2 * W)], ov)
    pltpu.sync_copy(d_hbm.at[pl.ds(b * HW, HW)], dv)

    iota = lax.iota(jnp.int32, LANES)
    nchunk = W // LANES

    def lg(ref, idx):
        return plsc.load_gather(ref, [idx])

    def rnd_int(v):
        return ((v + MAGIC) - MAGIC).astype(jnp.int32)

    def row_body(r, carry):
        base = r * W

        for i in range(nchunk):
            ci = iota + (base + i * LANES)
            uy = lg(uv, iota + (r * 2 * W + i * LANES))
            ux = lg(uv, iota + (r * 2 * W + W + i * LANES))
            y = uy * float(H - 1)
            x = ux * float(W - 1)
            d0 = lg(dv, rnd_int(y) * W + rnd_int(x))
            plsc.store_scatter(yb, [ci], y)
            plsc.store_scatter(xb, [ci], x)
            plsc.store_scatter(db, [ci], d0)

        def t_body(t, tc):
            for dirn in (1, -1):
                res = []
                for i in range(nchunk):
                    ci = iota + (base + i * LANES)
                    sj = ((iota + (i * LANES - dirn)) & (W - 1)) + base
                    cy, cx, cd = lg(yb, ci), lg(xb, ci), lg(db, ci)
                    sy, sx, sd = lg(yb, sj), lg(xb, sj), lg(db, sj)
                    m = sd < cd
                    res.append((ci,
                                jnp.where(m, sy, cy),
                                jnp.where(m, sx, cx),
                                jnp.where(m, sd, cd)))
                for ci, ny, nx, nd in res:
                    plsc.store_scatter(yb, [ci], ny)
                    plsc.store_scatter(xb, [ci], nx)
                    plsc.store_scatter(db, [ci], nd)

            for s in range(RADIUS):
                dr = t * RADIUS + s
                for i in range(nchunk):
                    ci = iota + (base + i * LANES)
                    ob = r * (RADIUS * ITERATIONS * 2 * W) + dr * 2 * W + i * LANES
                    offy = lg(ov, iota + ob)
                    offx = lg(ov, iota + (ob + W))
                    y, x, d0 = lg(yb, ci), lg(xb, ci), lg(db, ci)
                    ry = jnp.minimum(jnp.maximum(y + offy, 0.0), float(H - 1))
                    rx = jnp.minimum(jnp.maximum(x + offx, 0.0), float(W - 1))
                    rd = lg(dv, rnd_int(ry) * W + rnd_int(rx))
                    m = rd < d0
                    plsc.store_scatter(yb, [ci], jnp.where(m, ry, y))
                    plsc.store_scatter(xb, [ci], jnp.where(m, rx, x))
                    plsc.store_scatter(db, [ci], jnp.where(m, rd, d0))
            return tc

        lax.fori_loop(0, ITERATIONS, t_body, 0)

        for i in range(nchunk):
            ci = iota + (base + i * LANES)
            plsc.store_scatter(uv, [iota + (r * 2 * W + i * LANES)], lg(yb, ci))
            plsc.store_scatter(uv, [iota + (r * 2 * W + W + i * LANES)], lg(xb, ci))
        return carry

    lax.fori_loop(0, ROWS_PER_W, row_body, 0)

    pltpu.sync_copy(uv, out_hbm.at[pl.ds(rows0 * 2 * W, ROWS_PER_W * 2 * W)])


def _sc_loop(d_flat, u_flat, offs_flat):
    B = 2
    mesh = plsc.VectorSubcoreMesh(core_axis_name="c", subcore_axis_name="s")
    fn = functools.partial(
        pl.kernel,
        mesh=mesh,
        out_type=jax.ShapeDtypeStruct((B * H * 2 * W,), jnp.float32),
        scratch_types=[
            pltpu.VMEM((ROWS_PER_W * 2 * W,), jnp.float32),
            pltpu.VMEM((ROWS_PER_W * RADIUS * ITERATIONS * 2 * W,), jnp.float32),
            pltpu.VMEM((HW,), jnp.float32),
            pltpu.VMEM((ROWS_PER_W * W,), jnp.float32),
            pltpu.VMEM((ROWS_PER_W * W,), jnp.float32),
            pltpu.VMEM((ROWS_PER_W * W,), jnp.float32),
        ],
        compiler_params=pltpu.CompilerParams(needs_layout_passes=False),
    )(_sc_loop_body)
    return fn(d_flat, u_flat, offs_flat)


_K0 = (64467757, 2916123636)
_KS = [[2451885785, 2215112154], [2477523575, 3040475525],
       [3288317168, 3869482587], [3554626980, 3142212981],
       [1115580475, 397968394], [3965541470, 1466314410],
       [1329917820, 631477198], [3389937870, 4222981018],
       [845657194, 2085162261], [2019228077, 1846897043],
       [1878397639, 3912187480], [3118403341, 2122305751]]


def kernel(source, target):
    B, C, _, _ = source.shape

    dfield = _dfield(source, target)

    k0 = jax.random.wrap_key_data(jnp.array(_K0, dtype=jnp.uint32))
    u = jax.random.uniform(k0, (B, 2, H, W), dtype=jnp.float32)
    ks = jnp.array(_KS, dtype=jnp.uint32)
    offs = jax.vmap(
        lambda kk: jax.random.normal(jax.random.wrap_key_data(kk),
                                     (B, 2, H, W), dtype=jnp.float32))(ks)
    scales = (2.0 ** -jnp.arange(RADIUS, dtype=jnp.float32))
    offs = offs * jnp.tile(scales, ITERATIONS)[:, None, None, None, None]

    u_flat = jnp.transpose(u, (0, 2, 1, 3)).reshape(-1)
    offs_flat = jnp.transpose(offs, (1, 3, 0, 2, 4)).reshape(-1)

    out_flat = _sc_loop(dfield.reshape(-1), u_flat, offs_flat)
    return jnp.transpose(out_flat.reshape(B, H, 2, W), (0, 2, 1, 3))
